# Initial kernel scaffold; baseline (speedup 1.0000x reference)
import jax
import jax.numpy as jnp
from jax.experimental import pallas as pl


def kernel(x_querry, l, x_block, e_k, e_p):
    raise NotImplementedError("write your pallas kernel here")


# speedup vs baseline: 1.6224x; 1.6224x over previous
import functools

import jax
import jax.numpy as jnp
from jax import lax
from jax.experimental import pallas as pl
from jax.experimental.pallas import tpu as pltpu
from jax.experimental.pallas import tpu_sc as plsc

_B = 1024
_KEY_D = 768
_EMB_D = 768
_POOL = 100
_E_LEN = 20
_TOP_K = 2
_HALF = _E_LEN // 2
_ROW_W = _E_LEN * _EMB_D
_HALF_W = _HALF * _EMB_D
_NROWS = _B * _TOP_K

_NC = 2
_NS = 16
_NW = _NC * _NS
_BPW = _NROWS // _NW
_CHUNK = 8


def _select_body(xq_ref, ek_ref, idx_ref, cnt_ref):
    xq = xq_ref[...]
    ek = ek_ref[...]
    nk = ek / jnp.clip(jnp.sqrt(jnp.sum(ek * ek, axis=1, keepdims=True)), 1e-12)
    q = xq / jnp.clip(jnp.sqrt(jnp.sum(xq * xq, axis=1, keepdims=True)), 1e-12)
    cos = lax.dot_general(q, nk, (((1,), (1,)), ((), ())),
                          preferred_element_type=jnp.float32)
    iota = lax.broadcasted_iota(jnp.int32, (_B, _POOL), 1)
    m1 = jnp.max(cos, axis=1, keepdims=True)
    i1 = jnp.min(jnp.where(cos == m1, iota, _POOL), axis=1, keepdims=True)
    cos2 = jnp.where(iota == i1, -jnp.inf, cos)
    m2 = jnp.max(cos2, axis=1, keepdims=True)
    i2 = jnp.min(jnp.where(cos2 == m2, iota, _POOL), axis=1, keepdims=True)
    idx_ref[...] = jnp.concatenate([i1, i2], axis=1)
    cnt = ((iota == i1).astype(jnp.int32) + (iota == i2).astype(jnp.int32))
    cnt_ref[...] = jnp.sum(cnt, axis=0, keepdims=True)


def _select(x_querry, e_k):
    return pl.pallas_call(
        _select_body,
        out_shape=(
            jax.ShapeDtypeStruct((_B, _TOP_K), jnp.int32),
            jax.ShapeDtypeStruct((1, _POOL), jnp.int32),
        ),
    )(x_querry, e_k)


def _gather_body(ep_hbm, idx_hbm, ek_hbm, ev_hbm, idx_v, rows_v, sem, osem):
    wid = lax.axis_index("s") * _NC + lax.axis_index("c")
    base = wid * _BPW
    pltpu.sync_copy(idx_hbm.at[pl.ds(base, _BPW)], idx_v)

    def chunk(i, carry):
        f0 = base + i * _CHUNK
        pltpu.async_copy(
            ep_hbm.at[idx_v.at[pl.ds(i * _CHUNK, _CHUNK)]], rows_v, sem
        ).wait()
        ek_cp = pltpu.async_copy(
            rows_v.at[:, 0], ek_hbm.at[pl.ds(f0, _CHUNK)], osem)
        ev_cp = pltpu.async_copy(
            rows_v.at[:, 1], ev_hbm.at[pl.ds(f0, _CHUNK)], osem)
        ek_cp.wait()
        ev_cp.wait()
        return carry

    lax.fori_loop(0, _BPW // _CHUNK, chunk, 0)


@functools.partial(
    pl.kernel,
    mesh=plsc.VectorSubcoreMesh(core_axis_name="c", subcore_axis_name="s"),
    out_type=(
        jax.ShapeDtypeStruct((_NROWS, _HALF_W), jnp.float32),
        jax.ShapeDtypeStruct((_NROWS, _HALF_W), jnp.float32),
    ),
    scratch_types=[
        pltpu.VMEM((_BPW,), jnp.int32),
        pltpu.VMEM((_CHUNK, _TOP_K, _HALF_W), jnp.float32),
        pltpu.SemaphoreType.DMA,
        pltpu.SemaphoreType.DMA,
    ],
)
def _gather(ep_hbm, idx_hbm, ek_hbm, ev_hbm, idx_v, rows_v, sem, osem):
    _gather_body(ep_hbm, idx_hbm, ek_hbm, ev_hbm, idx_v, rows_v, sem, osem)


def kernel(x_querry, l, x_block, e_k, e_p):
    k_idx, cnt = _select(x_querry, e_k)
    idx_flat = k_idx.reshape(_NROWS)
    ep3 = e_p.reshape(_POOL, _TOP_K, _HALF_W)
    ek_flat, ev_flat = _gather(ep3, idx_flat)
    Ek = ek_flat.reshape(_B, _TOP_K * _HALF, _EMB_D)
    Ev = ev_flat.reshape(_B, _TOP_K * _HALF, _EMB_D)
    eval_count = cnt.reshape(_POOL)
    return (Ek, Ev, x_block, eval_count)


# speedup vs baseline: 1.6312x; 1.0054x over previous
import functools

import jax
import jax.numpy as jnp
from jax import lax
from jax.experimental import pallas as pl
from jax.experimental.pallas import tpu as pltpu
from jax.experimental.pallas import tpu_sc as plsc

_B = 1024
_KEY_D = 768
_EMB_D = 768
_POOL = 100
_E_LEN = 20
_TOP_K = 2
_HALF = _E_LEN // 2
_NROWS = _B * _TOP_K

_NC = 2
_NS = 16
_NW = _NC * _NS
_FPW = _NROWS // _NW
_CHUNK = 8
_NCHUNK = _FPW // _CHUNK


def _select_body(xq_ref, ek_ref, idx_ref, cnt_ref):
    xq = xq_ref[...]
    ek = ek_ref[...]
    nk = ek / jnp.clip(jnp.sqrt(jnp.sum(ek * ek, axis=1, keepdims=True)), 1e-12)
    q = xq / jnp.clip(jnp.sqrt(jnp.sum(xq * xq, axis=1, keepdims=True)), 1e-12)
    cos = lax.dot_general(q, nk, (((1,), (1,)), ((), ())),
                          preferred_element_type=jnp.float32)
    iota = lax.broadcasted_iota(jnp.int32, (_B, _POOL), 1)
    m1 = jnp.max(cos, axis=1, keepdims=True)
    i1 = jnp.min(jnp.where(cos == m1, iota, _POOL), axis=1, keepdims=True)
    cos2 = jnp.where(iota == i1, -jnp.inf, cos)
    m2 = jnp.max(cos2, axis=1, keepdims=True)
    i2 = jnp.min(jnp.where(cos2 == m2, iota, _POOL), axis=1, keepdims=True)
    idx_ref[...] = jnp.concatenate([i1, i2], axis=1)
    cnt = ((iota == i1).astype(jnp.int32) + (iota == i2).astype(jnp.int32))
    cnt_ref[...] = jnp.sum(cnt, axis=0, keepdims=True)


def _select(x_querry, e_k):
    return pl.pallas_call(
        _select_body,
        out_shape=(
            jax.ShapeDtypeStruct((_B, _TOP_K), jnp.int32),
            jax.ShapeDtypeStruct((1, _POOL), jnp.int32),
        ),
    )(x_querry, e_k)


def _gather_body(ep_hbm, idxp_hbm, ek_hbm, ev_hbm, idx8, rows_v, gsem, osem):
    wid = lax.axis_index("s") * _NC + lax.axis_index("c")

    def chunk(c, carry):
        g = wid * _NCHUNK + c
        b0 = wid * (_FPW // _TOP_K) + (_CHUNK // _TOP_K) * c
        pltpu.sync_copy(idxp_hbm.at[g], idx8)
        pltpu.async_copy(ep_hbm.at[idx8], rows_v, gsem).wait()
        cps = []
        for u in range(_CHUNK):
            b = b0 + u // _TOP_K
            r0 = (u % _TOP_K) * _HALF
            cps.append(pltpu.async_copy(
                rows_v.at[u, pl.ds(0, _HALF)],
                ek_hbm.at[b, pl.ds(r0, _HALF)], osem))
            cps.append(pltpu.async_copy(
                rows_v.at[u, pl.ds(_HALF, _HALF)],
                ev_hbm.at[b, pl.ds(r0, _HALF)], osem))
        for cp in cps:
            cp.wait()
        return carry

    lax.fori_loop(0, _NCHUNK, chunk, 0)


@functools.partial(
    pl.kernel,
    mesh=plsc.VectorSubcoreMesh(core_axis_name="c", subcore_axis_name="s"),
    out_type=(
        jax.ShapeDtypeStruct((_B, _E_LEN, _EMB_D), jnp.float32),
        jax.ShapeDtypeStruct((_B, _E_LEN, _EMB_D), jnp.float32),
    ),
    scratch_types=[
        pltpu.VMEM((_CHUNK,), jnp.int32),
        pltpu.VMEM((_CHUNK, _E_LEN, _EMB_D), jnp.float32),
        pltpu.SemaphoreType.DMA,
        pltpu.SemaphoreType.DMA,
    ],
    compiler_params=pltpu.CompilerParams(use_tc_tiling_on_sc=False),
)
def _gather(ep_hbm, idxp_hbm, ek_hbm, ev_hbm, idx8, rows_v, gsem, osem):
    _gather_body(ep_hbm, idxp_hbm, ek_hbm, ev_hbm, idx8, rows_v, gsem, osem)


def kernel(x_querry, l, x_block, e_k, e_p):
    k_idx, cnt = _select(x_querry, e_k)
    idx_pad = k_idx.reshape(_NROWS // _CHUNK, _CHUNK)
    Ek, Ev = _gather(e_p, idx_pad)
    eval_count = cnt.reshape(_POOL)
    return (Ek, Ev, x_block, eval_count)
